# baseline (speedup 1.0000x reference)
import jax
import jax.numpy as jnp
from jax import lax
from jax.experimental import pallas as pl
from jax.experimental.pallas import tpu as pltpu
from jax.experimental.pallas import tpu_sc as plsc

_BATCH = 4096
_HIST = 200
_EMB_DIM = 64
_B = _BATCH * _HIST
_NW = 32
_NROWS = 1000001
_NPAIR = 500004

_mesh = plsc.VectorSubcoreMesh(core_axis_name="c", subcore_axis_name="s")


def _wid():
    return lax.axis_index("s") * 2 + lax.axis_index("c")


def _move_pairs(src, dst, n_pairs, src_off=0, dst_off=0):
    def step(u, carry):
        for k in range(4):
            dst[dst_off + u, pl.ds(16 * k, 16)] = (
                src[src_off + 2 * u, pl.ds(16 * k, 16)])
            dst[dst_off + u, pl.ds(64 + 16 * k, 16)] = (
                src[src_off + 2 * u + 1, pl.ds(16 * k, 16)])
        return carry

    lax.fori_loop(0, n_pairs, step, 0)


def _move_halves(src, dst, n_pairs):
    def step(u, carry):
        for k in range(4):
            dst[2 * u, pl.ds(16 * k, 16)] = src[u, pl.ds(16 * k, 16)]
            dst[2 * u + 1, pl.ds(16 * k, 16)] = src[u, pl.ds(64 + 16 * k, 16)]
        return carry

    lax.fori_loop(0, n_pairs, step, 0)


_D_CHUNK = 480
_D_NFULL = 2083
_D_TAIL0 = _D_NFULL * _D_CHUNK


def _depad_body(table_hbm, tp_hbm, buf, buf2):
    wid = _wid()
    nch = jnp.where(wid < 3, 66, 65)
    base = wid * 65 + jnp.minimum(wid, 3)

    def step(c, carry):
        r0 = pl.multiple_of((base + c) * _D_CHUNK, 8)
        p0 = pl.multiple_of((base + c) * (_D_CHUNK // 2), 8)
        pltpu.sync_copy(table_hbm.at[pl.ds(r0, _D_CHUNK)], buf)
        _move_pairs(buf, buf2, _D_CHUNK // 2)
        pltpu.sync_copy(buf2, tp_hbm.at[pl.ds(p0, _D_CHUNK // 2)])
        return carry

    lax.fori_loop(0, nch, step, 0)

    @pl.when(wid == 31)
    def _():
        pltpu.sync_copy(table_hbm.at[pl.ds(_D_TAIL0, 160)],
                        buf.at[pl.ds(0, 160)])
        _move_pairs(buf, buf2, 80)
        pltpu.sync_copy(buf2.at[pl.ds(0, 80)],
                        tp_hbm.at[pl.ds(_D_TAIL0 // 2, 80)])
        pltpu.sync_copy(table_hbm.at[pl.ds(_NROWS - 1, 1)],
                        buf.at[pl.ds(0, 1)])

        def last(k, carry):
            buf2[0, pl.ds(16 * k, 16)] = buf[0, pl.ds(16 * k, 16)]
            return carry

        lax.fori_loop(0, 4, last, 0)
        pltpu.sync_copy(buf2.at[pl.ds(0, 1)],
                        tp_hbm.at[pl.ds((_NROWS - 1) // 2, 1)])


_depad = pl.kernel(
    _depad_body,
    out_type=jax.ShapeDtypeStruct((_NPAIR, 128), jnp.float32),
    mesh=_mesh,
    scratch_types=[
        pltpu.VMEM((_D_CHUNK, _EMB_DIM), jnp.float32),
        pltpu.VMEM((_D_CHUNK // 2, 128), jnp.float32),
    ],
)


_B_PER_W = _B // _NW
_CHUNK = 800
_N_CHUNKS = _B_PER_W // _CHUNK


def _gather_body(x_hbm, table_hbm, out_hbm, idx_v, rows0, rows1, gsem0, gsem1,
                 osem0, osem1):
    base = _wid() * _B_PER_W

    pltpu.sync_copy(x_hbm.at[pl.ds(base, _B_PER_W)], idx_v)

    rows = (rows0, rows1)
    gsem = (gsem0, gsem1)
    osem = (osem0, osem1)

    def gather(g, buf):
        return pltpu.make_async_copy(
            table_hbm.at[idx_v.at[pl.ds(g * _CHUNK, _CHUNK)]],
            rows[buf], gsem[buf])

    def flush(g, buf):
        return pltpu.make_async_copy(
            rows[buf], out_hbm.at[pl.ds(base + g * _CHUNK, _CHUNK)],
            osem[buf])

    gather(0, 0).start()

    def step(t, carry):
        g0 = 2 * t
        g1 = g0 + 1

        @pl.when(t > 0)
        def _():
            flush(g0 - 1, 1).wait()

        gather(g1, 1).start()
        gather(g0, 0).wait()
        flush(g0, 0).start()

        flush(g0, 0).wait()

        @pl.when(t < _N_CHUNKS // 2 - 1)
        def _():
            gather(g0 + 2, 0).start()

        gather(g1, 1).wait()
        flush(g1, 1).start()
        return carry

    lax.fori_loop(0, _N_CHUNKS // 2, step, 0)
    flush(_N_CHUNKS - 1, 1).wait()


_gather = pl.kernel(
    _gather_body,
    out_type=jax.ShapeDtypeStruct((_B, _EMB_DIM), jnp.float32),
    mesh=_mesh,
    scratch_types=[
        pltpu.VMEM((_B_PER_W,), jnp.int32),
        pltpu.VMEM((_CHUNK, _EMB_DIM), jnp.float32),
        pltpu.VMEM((_CHUNK, _EMB_DIM), jnp.float32),
        pltpu.SemaphoreType.DMA,
        pltpu.SemaphoreType.DMA,
        pltpu.SemaphoreType.DMA,
        pltpu.SemaphoreType.DMA,
    ],
    compiler_params=pltpu.CompilerParams(use_tc_tiling_on_sc=False),
)


_P_CHUNK = 400
_P_NCH = _B_PER_W // _P_CHUNK


def _pad_out_body(emb_hbm, out_hbm, bufc, bufd):
    wid = _wid()
    base = wid * _B_PER_W

    def step(g, carry):
        p0 = pl.multiple_of(base // 2 + g * (_P_CHUNK // 2), 8)
        pltpu.sync_copy(emb_hbm.at[pl.ds(p0, _P_CHUNK // 2)], bufc)

        def unpack(u, carry2):
            for b in range(2):
                for k in range(4):
                    bufd[b, 2 * u, pl.ds(16 * k, 16)] = (
                        bufc[100 * b + u, pl.ds(16 * k, 16)])
                    bufd[b, 2 * u + 1, pl.ds(16 * k, 16)] = (
                        bufc[100 * b + u, pl.ds(64 + 16 * k, 16)])
            return carry2

        lax.fori_loop(0, 100, unpack, 0)
        pltpu.sync_copy(bufd, out_hbm.at[pl.ds(wid * 128 + 2 * g, 2)])
        return carry

    lax.fori_loop(0, _P_NCH, step, 0)


_pad_out = pl.kernel(
    _pad_out_body,
    out_type=jax.ShapeDtypeStruct((_BATCH, _HIST, _EMB_DIM), jnp.float32),
    mesh=_mesh,
    scratch_types=[
        pltpu.VMEM((_P_CHUNK // 2, 128), jnp.float32),
        pltpu.VMEM((2, _HIST, _EMB_DIM), jnp.float32),
    ],
)


@jax.jit
def kernel(x, table):
    xf = x.reshape(-1).astype(jnp.int32)
    tp = _depad(table)
    emb = _gather(xf, tp.reshape(_NPAIR * 2, _EMB_DIM))
    return _pad_out(emb.reshape(_B // 2, 128))


# speedup vs baseline: 1.1109x; 1.1109x over previous
import jax
import jax.numpy as jnp
from jax import lax
from jax.experimental import pallas as pl
from jax.experimental.pallas import tpu as pltpu
from jax.experimental.pallas import tpu_sc as plsc

_BATCH = 4096
_HIST = 200
_EMB_DIM = 64
_B = _BATCH * _HIST
_NW = 32
_NROWS = 1000001
_NPAIR = 500004

_mesh = plsc.VectorSubcoreMesh(core_axis_name="c", subcore_axis_name="s")


_D_BLK = 2048


def _depad_body(t_ref, o_ref):
    z = t_ref[...].reshape(_D_BLK // 2, 2, _EMB_DIM)
    o_ref[:, :_EMB_DIM] = z[:, 0, :]
    o_ref[:, _EMB_DIM:] = z[:, 1, :]


_depad = pl.pallas_call(
    _depad_body,
    grid=((_NROWS + _D_BLK - 1) // _D_BLK,),
    in_specs=[pl.BlockSpec((_D_BLK, _EMB_DIM), lambda i: (i, 0))],
    out_specs=pl.BlockSpec((_D_BLK // 2, 128), lambda i: (i, 0)),
    out_shape=jax.ShapeDtypeStruct((_NPAIR, 128), jnp.float32),
)


_B_PER_W = _B // _NW
_CHUNK = 800
_N_CHUNKS = _B_PER_W // _CHUNK


def _gather_body(x_hbm, table_hbm, out_hbm, idx_v, rows0, rows1, gsem0, gsem1,
                 osem0, osem1):
    base = (lax.axis_index("s") * 2 + lax.axis_index("c")) * _B_PER_W

    pltpu.sync_copy(x_hbm.at[pl.ds(base, _B_PER_W)], idx_v)

    rows = (rows0, rows1)
    gsem = (gsem0, gsem1)
    osem = (osem0, osem1)

    def gather(g, buf):
        return pltpu.make_async_copy(
            table_hbm.at[idx_v.at[pl.ds(g * _CHUNK, _CHUNK)]],
            rows[buf], gsem[buf])

    def flush(g, buf):
        return pltpu.make_async_copy(
            rows[buf], out_hbm.at[pl.ds(base + g * _CHUNK, _CHUNK)],
            osem[buf])

    gather(0, 0).start()

    def step(t, carry):
        g0 = 2 * t
        g1 = g0 + 1

        @pl.when(t > 0)
        def _():
            flush(g0 - 1, 1).wait()

        gather(g1, 1).start()
        gather(g0, 0).wait()
        flush(g0, 0).start()

        flush(g0, 0).wait()

        @pl.when(t < _N_CHUNKS // 2 - 1)
        def _():
            gather(g0 + 2, 0).start()

        gather(g1, 1).wait()
        flush(g1, 1).start()
        return carry

    lax.fori_loop(0, _N_CHUNKS // 2, step, 0)
    flush(_N_CHUNKS - 1, 1).wait()


_gather = pl.kernel(
    _gather_body,
    out_type=jax.ShapeDtypeStruct((_B, _EMB_DIM), jnp.float32),
    mesh=_mesh,
    scratch_types=[
        pltpu.VMEM((_B_PER_W,), jnp.int32),
        pltpu.VMEM((_CHUNK, _EMB_DIM), jnp.float32),
        pltpu.VMEM((_CHUNK, _EMB_DIM), jnp.float32),
        pltpu.SemaphoreType.DMA,
        pltpu.SemaphoreType.DMA,
        pltpu.SemaphoreType.DMA,
        pltpu.SemaphoreType.DMA,
    ],
    compiler_params=pltpu.CompilerParams(use_tc_tiling_on_sc=False),
)


_P_BLK = 8


def _pad_out_body(e_ref, o_ref):
    x = e_ref[...]
    z = jnp.concatenate(
        [x[:, None, :_EMB_DIM], x[:, None, _EMB_DIM:]], axis=1)
    o_ref[...] = z.reshape(_P_BLK, _HIST, _EMB_DIM)


_pad_out = pl.pallas_call(
    _pad_out_body,
    grid=(_BATCH // _P_BLK,),
    in_specs=[pl.BlockSpec((_P_BLK * _HIST // 2, 128), lambda i: (i, 0))],
    out_specs=pl.BlockSpec((_P_BLK, _HIST, _EMB_DIM), lambda i: (i, 0, 0)),
    out_shape=jax.ShapeDtypeStruct((_BATCH, _HIST, _EMB_DIM), jnp.float32),
)


@jax.jit
def kernel(x, table):
    xf = x.reshape(-1).astype(jnp.int32)
    tp = _depad(table)
    emb = _gather(xf, tp.reshape(_NPAIR * 2, _EMB_DIM))
    return _pad_out(emb.reshape(_B // 2, 128))


# speedup vs baseline: 1.7125x; 1.5416x over previous
import jax
import jax.numpy as jnp
from jax import lax
from jax.experimental import pallas as pl
from jax.experimental.pallas import tpu as pltpu
from jax.experimental.pallas import tpu_sc as plsc

_BATCH = 4096
_HIST = 200
_EMB_DIM = 64
_B = _BATCH * _HIST
_NW = 32
_B_PER_W = _B // _NW
_CHUNK = 800
_N_CHUNKS = _B_PER_W // _CHUNK

_mesh = plsc.VectorSubcoreMesh(core_axis_name="c", subcore_axis_name="s")


def _emb_body(x_hbm, table_hbm, out_hbm, idx_v, rows0, rows1, gsem0, gsem1,
              osem0, osem1):
    wid = lax.axis_index("s") * 2 + lax.axis_index("c")
    base = wid * _B_PER_W

    pltpu.sync_copy(x_hbm.at[pl.ds(base, _B_PER_W)], idx_v)

    rows = (rows0, rows1)
    gsem = (gsem0, gsem1)
    osem = (osem0, osem1)

    def gather(g, buf):
        return pltpu.make_async_copy(
            table_hbm.at[idx_v.at[pl.ds(g * _CHUNK, _CHUNK)]],
            rows[buf], gsem[buf])

    def flush(g, buf):
        return pltpu.make_async_copy(
            rows[buf], out_hbm.at[pl.ds(base + g * _CHUNK, _CHUNK)],
            osem[buf])

    gather(0, 0).start()

    def step(t, carry):
        g0 = 2 * t
        g1 = g0 + 1

        @pl.when(t > 0)
        def _():
            flush(g0 - 1, 1).wait()

        gather(g1, 1).start()
        gather(g0, 0).wait()
        flush(g0, 0).start()

        flush(g0, 0).wait()

        @pl.when(t < _N_CHUNKS // 2 - 1)
        def _():
            gather(g0 + 2, 0).start()

        gather(g1, 1).wait()
        flush(g1, 1).start()
        return carry

    lax.fori_loop(0, _N_CHUNKS // 2, step, 0)
    flush(_N_CHUNKS - 1, 1).wait()


_emb = pl.kernel(
    _emb_body,
    out_type=jax.ShapeDtypeStruct((_B, _EMB_DIM), jnp.float32),
    mesh=_mesh,
    scratch_types=[
        pltpu.VMEM((_B_PER_W,), jnp.int32),
        pltpu.VMEM((_CHUNK, _EMB_DIM), jnp.float32),
        pltpu.VMEM((_CHUNK, _EMB_DIM), jnp.float32),
        pltpu.SemaphoreType.DMA,
        pltpu.SemaphoreType.DMA,
        pltpu.SemaphoreType.DMA,
        pltpu.SemaphoreType.DMA,
    ],
    compiler_params=pltpu.CompilerParams(
        use_tc_tiling_on_sc=False, skip_device_barrier=True),
)


@jax.jit
def kernel(x, table):
    xf = x.reshape(-1).astype(jnp.int32)
    out = _emb(xf, table)
    return out.reshape(_BATCH, _HIST, _EMB_DIM)
